# Initial kernel scaffold; baseline (speedup 1.0000x reference)
import jax
import jax.numpy as jnp
from jax.experimental import pallas as pl


def kernel(inputs):
    raise NotImplementedError("write your pallas kernel here")


# speedup vs baseline: 17.0757x; 17.0757x over previous
import functools

import jax
import jax.numpy as jnp
from jax import lax
from jax.experimental import pallas as pl
from jax.experimental.pallas import tpu as pltpu
from jax.experimental.pallas import tpu_sc as plsc

B, S, C, K = 4, 4096, 1024, 16
NW = 32
CPW = B * C // NW
NG = CPW // 16
CBLK = C // CPW


def _sort16_desc_pairs():
    pairs = []
    k = 2
    while k <= 16:
        j = k // 2
        while j >= 1:
            for i in range(16):
                l = i ^ j
                if l > i:
                    pairs.append((i, l, (i & k) == 0))
            j //= 2
        k *= 2
    return pairs


_SORT16 = _sort16_desc_pairs()
_MERGE16 = [(i, i + j) for j in (8, 4, 2, 1) for i in range(16)
            if (i & j) == 0 and i + j < 16]


def _sort16_desc(rows):
    rows = list(rows)
    for i, l, up in _SORT16:
        a, b = rows[i], rows[l]
        if up:
            rows[i], rows[l] = jnp.maximum(a, b), jnp.minimum(a, b)
        else:
            rows[i], rows[l] = jnp.minimum(a, b), jnp.maximum(a, b)
    return rows


def _bitonic_merge16_desc(rows):
    rows = list(rows)
    for i, l in _MERGE16:
        a, b = rows[i], rows[l]
        rows[i], rows[l] = jnp.maximum(a, b), jnp.minimum(a, b)
    return rows


def _sc_topk(x):
    mesh = plsc.VectorSubcoreMesh(core_axis_name="c", subcore_axis_name="s")

    @functools.partial(
        pl.kernel,
        out_type=jax.ShapeDtypeStruct((NW, K, CPW), jnp.float32),
        mesh=mesh,
        scratch_types=[
            pltpu.VMEM((S, 16), jnp.float32),
            pltpu.VMEM((K, CPW), jnp.float32),
        ],
        compiler_params=pltpu.CompilerParams(use_tc_tiling_on_sc=False),
    )
    def k(x_hbm, out_hbm, buf, outv):
        cid = lax.axis_index("c")
        sid = lax.axis_index("s")
        wid = sid * 2 + cid
        b = wid // CBLK
        c0 = (wid % CBLK) * CPW

        for g in range(NG):
            pltpu.sync_copy(x_hbm.at[b, :, pl.ds(c0 + g * 16, 16)], buf)

            def block(i, R):
                base = i * 16
                X = [buf[base + j, :] for j in range(16)]
                X = _sort16_desc(X)
                merged = [jnp.maximum(R[j], X[15 - j]) for j in range(16)]
                return tuple(_bitonic_merge16_desc(merged))

            neg_inf = jnp.full((16,), -jnp.inf, jnp.float32)
            R = lax.fori_loop(0, S // 16, block, (neg_inf,) * 16)
            for j in range(16):
                outv[j, pl.ds(g * 16, 16)] = R[j]

        pltpu.sync_copy(outv, out_hbm.at[wid])

    return k(x)


@jax.jit
def kernel(inputs):
    out = _sc_topk(inputs)
    out = out.reshape(B, CBLK, K, CPW)
    out = jnp.transpose(out, (0, 1, 3, 2))
    return out.reshape(B, C * K)
